# baseline (speedup 1.0000x reference)
import functools

import jax
import jax.numpy as jnp
from jax import lax
from jax.experimental import pallas as pl
from jax.experimental.pallas import tpu as pltpu
from jax.experimental.pallas import tpu_sc as plsc

B = 16384
D = 500
H1 = 64
NC = 2
NS = 16
NW = NC * NS
ROWS_PER_W = B // NW
CHUNK = 64
NCHUNK = ROWS_PER_W // CHUNK

@functools.cache
def _make_sc_gather():
    mesh = plsc.VectorSubcoreMesh(core_axis_name="c", subcore_axis_name="s")

    @functools.partial(
        pl.kernel,
        mesh=mesh,
        out_type=(
            jax.ShapeDtypeStruct((B, D), jnp.float32),
            jax.ShapeDtypeStruct((B, D), jnp.float32),
        ),
        scratch_types=[
            pltpu.VMEM((NCHUNK, CHUNK), jnp.int32),
            pltpu.VMEM((NCHUNK, CHUNK), jnp.int32),
            pltpu.VMEM((CHUNK, D), jnp.float32),
            pltpu.VMEM((CHUNK, D), jnp.float32),
            pltpu.SemaphoreType.DMA,
            pltpu.SemaphoreType.DMA,
        ],
        compiler_params=pltpu.CompilerParams(use_tc_tiling_on_sc=False),
    )
    def _sc_gather(uid_hbm, iid_hbm, utab_hbm, itab_hbm, uout_hbm, iout_hbm,
                   uidx_v, iidx_v, urows_v, irows_v, usem, isem):
        wid = lax.axis_index("s") * NC + lax.axis_index("c")
        pltpu.sync_copy(uid_hbm.at[wid], uidx_v)
        pltpu.sync_copy(iid_hbm.at[wid], iidx_v)
        for j in range(NCHUNK):
            base = wid * ROWS_PER_W + j * CHUNK
            ucp = pltpu.async_copy(utab_hbm.at[uidx_v.at[j]], urows_v, usem)
            icp = pltpu.async_copy(itab_hbm.at[iidx_v.at[j]], irows_v, isem)
            ucp.wait()
            pltpu.sync_copy(urows_v, uout_hbm.at[pl.ds(base, CHUNK)])
            icp.wait()
            pltpu.sync_copy(irows_v, iout_hbm.at[pl.ds(base, CHUNK)])

    return _sc_gather


def _tc_dense_body(u_ref, i_ref, wu_ref, wi_ref, bu_ref, bi_ref, wp_ref, bp_ref,
                   o_ref):
    u = jnp.dot(u_ref[...], wu_ref[...], preferred_element_type=jnp.float32)
    u = jnp.maximum(u + bu_ref[...], 0.0)
    i = jnp.dot(i_ref[...], wi_ref[...], preferred_element_type=jnp.float32)
    i = jnp.maximum(i + bi_ref[...], 0.0)
    r = jnp.dot(u, wp_ref[:H1, :], preferred_element_type=jnp.float32)
    r = r + jnp.dot(i, wp_ref[H1:, :], preferred_element_type=jnp.float32)
    o_ref[...] = r + bp_ref[...]


BB = 2048


def _tc_dense(u_docs, i_docs, wu, wi, bu, bi, wp, bp):
    grid = (B // BB,)
    return pl.pallas_call(
        _tc_dense_body,
        grid=grid,
        in_specs=[
            pl.BlockSpec((BB, D), lambda b: (b, 0)),
            pl.BlockSpec((BB, D), lambda b: (b, 0)),
            pl.BlockSpec((D, H1), lambda b: (0, 0)),
            pl.BlockSpec((D, H1), lambda b: (0, 0)),
            pl.BlockSpec((1, H1), lambda b: (0, 0)),
            pl.BlockSpec((1, H1), lambda b: (0, 0)),
            pl.BlockSpec((2 * H1, 1), lambda b: (0, 0)),
            pl.BlockSpec((1, 1), lambda b: (0, 0)),
        ],
        out_specs=pl.BlockSpec((BB, 1), lambda b: (b, 0)),
        out_shape=jax.ShapeDtypeStruct((B, 1), jnp.float32),
    )(u_docs, i_docs, wu, wi, bu, bi, wp, bp)


def kernel(batch_uid, batch_iid, uid_userDoc, iid_itemDoc, userFC_W, userFC_b,
           itemFC_W, itemFC_b, pred_W, pred_b):
    u_docs = jnp.take(uid_userDoc, batch_uid, axis=0)
    i_docs = jnp.take(iid_itemDoc, batch_iid, axis=0)
    out = _tc_dense(u_docs, i_docs, userFC_W, itemFC_W,
                    userFC_b.reshape(1, H1), itemFC_b.reshape(1, H1),
                    pred_W, pred_b.reshape(1, 1))
    return out


# speedup vs baseline: 3.9409x; 3.9409x over previous
import functools

import jax
import jax.numpy as jnp
from jax import lax
from jax.experimental import pallas as pl
from jax.experimental.pallas import tpu as pltpu
from jax.experimental.pallas import tpu_sc as plsc

B = 16384
D = 500
DA = 384
DT = D - DA
H1 = 64
NC = 2
NS = 16
NW = NC * NS
ROWS_PER_W = B // NW
CHUNK = 64
NCHUNK = ROWS_PER_W // CHUNK


@functools.cache
def _make_sc_gather():
    mesh = plsc.VectorSubcoreMesh(core_axis_name="c", subcore_axis_name="s")

    @functools.partial(
        pl.kernel,
        mesh=mesh,
        out_type=(
            jax.ShapeDtypeStruct((B, D), jnp.float32),
            jax.ShapeDtypeStruct((B, D), jnp.float32),
        ),
        scratch_types=[
            pltpu.VMEM((NCHUNK, CHUNK), jnp.int32),
            pltpu.VMEM((NCHUNK, CHUNK), jnp.int32),
            pltpu.VMEM((CHUNK, D), jnp.float32),
            pltpu.VMEM((CHUNK, D), jnp.float32),
            pltpu.SemaphoreType.DMA,
            pltpu.SemaphoreType.DMA,
            pltpu.SemaphoreType.DMA,
        ],
    )
    def _sc_gather(uid_hbm, iid_hbm, utab_hbm, itab_hbm, uout_hbm, iout_hbm,
                   uidx_v, iidx_v, urows_v, irows_v, usem, isem, tsem):
        wid = lax.axis_index("s") * NC + lax.axis_index("c")
        pltpu.sync_copy(uid_hbm.at[wid], uidx_v)
        pltpu.sync_copy(iid_hbm.at[wid], iidx_v)
        for j in range(NCHUNK):
            base = wid * ROWS_PER_W + j * CHUNK
            ucp = pltpu.async_copy(
                utab_hbm.at[uidx_v.at[j], pl.ds(0, DA)],
                urows_v.at[:, pl.ds(0, DA)], usem)
            icp = pltpu.async_copy(
                itab_hbm.at[iidx_v.at[j], pl.ds(0, DA)],
                irows_v.at[:, pl.ds(0, DA)], isem)

            def tails(r16, _):
                vu = uidx_v[j, pl.ds(r16 * 16, 16)]
                vi = iidx_v[j, pl.ds(r16 * 16, 16)]
                for lane in range(16):
                    r = r16 * 16 + lane
                    pltpu.async_copy(
                        utab_hbm.at[pl.ds(vu[lane], 1), pl.ds(DA, DT)],
                        urows_v.at[pl.ds(r, 1), pl.ds(DA, DT)], tsem)
                    pltpu.async_copy(
                        itab_hbm.at[pl.ds(vi[lane], 1), pl.ds(DA, DT)],
                        irows_v.at[pl.ds(r, 1), pl.ds(DA, DT)], tsem)
                return 0

            lax.fori_loop(0, CHUNK // 16, tails, 0)

            def drain(r, _):
                pltpu.make_async_copy(
                    utab_hbm.at[pl.ds(0, 1), pl.ds(DA, DT)],
                    urows_v.at[pl.ds(0, 1), pl.ds(DA, DT)], tsem).wait()
                pltpu.make_async_copy(
                    utab_hbm.at[pl.ds(0, 1), pl.ds(DA, DT)],
                    irows_v.at[pl.ds(0, 1), pl.ds(DA, DT)], tsem).wait()
                return 0

            lax.fori_loop(0, CHUNK, drain, 0)
            ucp.wait()
            icp.wait()
            pltpu.sync_copy(urows_v, uout_hbm.at[pl.ds(base, CHUNK)])
            pltpu.sync_copy(irows_v, iout_hbm.at[pl.ds(base, CHUNK)])

    return _sc_gather


def _tc_dense_body(u_ref, i_ref, wu_ref, wi_ref, bu_ref, bi_ref, wp_ref, bp_ref,
                   o_ref):
    u = jnp.dot(u_ref[...], wu_ref[...], preferred_element_type=jnp.float32)
    u = jnp.maximum(u + bu_ref[...], 0.0)
    i = jnp.dot(i_ref[...], wi_ref[...], preferred_element_type=jnp.float32)
    i = jnp.maximum(i + bi_ref[...], 0.0)
    r = jnp.dot(u, wp_ref[:H1, :], preferred_element_type=jnp.float32)
    r = r + jnp.dot(i, wp_ref[H1:, :], preferred_element_type=jnp.float32)
    o_ref[...] = r + bp_ref[...]


BB = 2048


def _tc_dense(u_docs, i_docs, wu, wi, bu, bi, wp, bp):
    grid = (B // BB,)
    return pl.pallas_call(
        _tc_dense_body,
        grid=grid,
        in_specs=[
            pl.BlockSpec((BB, D), lambda b: (b, 0)),
            pl.BlockSpec((BB, D), lambda b: (b, 0)),
            pl.BlockSpec((D, H1), lambda b: (0, 0)),
            pl.BlockSpec((D, H1), lambda b: (0, 0)),
            pl.BlockSpec((1, H1), lambda b: (0, 0)),
            pl.BlockSpec((1, H1), lambda b: (0, 0)),
            pl.BlockSpec((2 * H1, 1), lambda b: (0, 0)),
            pl.BlockSpec((1, 1), lambda b: (0, 0)),
        ],
        out_specs=pl.BlockSpec((BB, 1), lambda b: (b, 0)),
        out_shape=jax.ShapeDtypeStruct((B, 1), jnp.float32),
    )(u_docs, i_docs, wu, wi, bu, bi, wp, bp)


def kernel(batch_uid, batch_iid, uid_userDoc, iid_itemDoc, userFC_W, userFC_b,
           itemFC_W, itemFC_b, pred_W, pred_b):
    uid = batch_uid.astype(jnp.int32).reshape(NW, NCHUNK, CHUNK)
    iid = batch_iid.astype(jnp.int32).reshape(NW, NCHUNK, CHUNK)
    u_docs, i_docs = _make_sc_gather()(uid, iid, uid_userDoc, iid_itemDoc)
    out = _tc_dense(u_docs, i_docs, userFC_W, itemFC_W,
                    userFC_b.reshape(1, H1), itemFC_b.reshape(1, H1),
                    pred_W, pred_b.reshape(1, 1))
    return out
